# baseline (speedup 1.0000x reference)
import functools

import jax
import jax.numpy as jnp
from jax import lax
from jax.experimental import pallas as pl
from jax.experimental.pallas import tpu as pltpu
from jax.experimental.pallas import tpu_sc as plsc

VOCAB = 1000000
DIM = 64
BATCH = 16384
CTX = 20
NEG = 20

NC = 2
NS = 16
NW = NC * NS
B_PER_W = BATCH // NW
R = 16
NCHUNK = B_PER_W // R
CTX_N = R * CTX
NEG_N = R * NEG
GSUB = 64
NSUB = CTX_N // GSUB
FILL = 1.0e9


def _unpack4(ref, row, base):
    out = []
    for half in range(2):
        a, b = plsc.unpack(ref[row, pl.ds(base + half * 32, 32)],
                           format=plsc.PackFormat.INTERLEAVED)
        out += [a, b]
    return out


def _sc_dots(cat_tbl, ctx3d, tgt_flat, neg3d):
    mesh = plsc.VectorSubcoreMesh(core_axis_name="c", subcore_axis_name="s")

    @functools.partial(
        pl.kernel,
        mesh=mesh,
        out_type=jax.ShapeDtypeStruct((BATCH, 32), jnp.float32),
        compiler_params=pltpu.CompilerParams(
            needs_layout_passes=False, use_tc_tiling_on_sc=False),
        scratch_types=[
            pltpu.VMEM((B_PER_W * CTX // GSUB, GSUB), jnp.int32),
            pltpu.VMEM((B_PER_W * NEG // GSUB, GSUB), jnp.int32),
            pltpu.VMEM((B_PER_W,), jnp.int32),
            pltpu.VMEM((CTX_N, 2 * DIM), jnp.bfloat16),
            pltpu.VMEM((NEG_N, 2 * DIM), jnp.bfloat16),
            pltpu.VMEM((R, 2 * DIM), jnp.bfloat16),
            pltpu.VMEM((R, 32), jnp.float32),
            pltpu.SemaphoreType.DMA,
        ],
    )
    def k(tbl_hbm, ctx_hbm, tgt_hbm, neg_hbm, dots_o,
          ctx_idx, neg_idx, tgt_idx, ctx_rows, neg_rows, tgt_rows,
          dots_v, sem):
        wid = lax.axis_index("s") * NC + lax.axis_index("c")
        lane = lax.iota(jnp.int32, 16)
        pltpu.sync_copy(ctx_hbm.at[wid], ctx_idx)
        pltpu.sync_copy(neg_hbm.at[wid], neg_idx)
        pltpu.sync_copy(tgt_hbm.at[pl.ds(wid * B_PER_W, B_PER_W)], tgt_idx)

        def chunk_body(i, _):
            row0 = wid * B_PER_W + i * R
            cps = []
            for s in range(NSUB):
                cps.append(pltpu.async_copy(
                    tbl_hbm.at[ctx_idx.at[i * NSUB + s]],
                    ctx_rows.at[pl.ds(s * GSUB, GSUB)], sem))
                cps.append(pltpu.async_copy(
                    tbl_hbm.at[neg_idx.at[i * NSUB + s]],
                    neg_rows.at[pl.ds(s * GSUB, GSUB)], sem))
            cps.append(pltpu.async_copy(
                tbl_hbm.at[tgt_idx.at[pl.ds(i * R, R)]], tgt_rows, sem))
            for cp in cps:
                cp.wait()

            def row_body(r, _):
                h = _unpack4(ctx_rows, r * CTX, 0)
                for c in range(1, CTX):
                    v = _unpack4(ctx_rows, r * CTX + c, 0)
                    h = [x + y for x, y in zip(h, v)]
                h = [x * (1.0 / CTX) for x in h]
                nh = [-x for x in h]
                t = _unpack4(tgt_rows, r, DIM)
                acc = t[0] * h[0]
                for d in range(1, 4):
                    acc = acc + t[d] * h[d]
                v0 = jnp.where(lane == 0, jnp.sum(acc), jnp.full((16,), FILL))
                v1 = jnp.full((16,), FILL)
                for j in range(NEG):
                    g = _unpack4(neg_rows, r * NEG + j, DIM)
                    acc = g[0] * nh[0]
                    for d in range(1, 4):
                        acc = acc + g[d] * nh[d]
                    dot = jnp.sum(acc)
                    if j + 1 < 16:
                        v0 = jnp.where(lane == (j + 1), dot, v0)
                    else:
                        v1 = jnp.where(lane == (j + 1 - 16), dot, v1)
                dots_v[r, pl.ds(0, 16)] = v0
                dots_v[r, pl.ds(16, 16)] = v1
                return 0

            lax.fori_loop(0, R, row_body, 0)
            pltpu.sync_copy(dots_v, dots_o.at[pl.ds(row0, R)])
            return 0

        lax.fori_loop(0, NCHUNK, chunk_body, 0)

    return k(cat_tbl, ctx3d, tgt_flat, neg3d)


def _tc_loss(dots2d):
    def body(dots_ref, out_ref):
        s = -jnp.sum(jax.nn.log_sigmoid(dots_ref[...])) / BATCH
        out_ref[...] = jnp.full((1, 1), s, dtype=jnp.float32)

    out = pl.pallas_call(
        body,
        out_shape=jax.ShapeDtypeStruct((1, 1), jnp.float32),
    )(dots2d)
    return out[0, 0]


def kernel(in_table, out_table, contexts, targets, negative_sampling):
    cat_tbl = jnp.concatenate(
        [in_table, out_table], axis=1).astype(jnp.bfloat16)
    ctx3d = contexts.astype(jnp.int32).reshape(NW, B_PER_W * CTX // GSUB, GSUB)
    neg3d = negative_sampling.astype(jnp.int32).reshape(
        NW, B_PER_W * NEG // GSUB, GSUB)
    tgt_flat = targets.astype(jnp.int32).reshape(BATCH)
    dots = _sc_dots(cat_tbl, ctx3d, tgt_flat, neg3d)
    return _tc_loss(dots.reshape(BATCH * 32 // 128, 128))


# speedup vs baseline: 1.6270x; 1.6270x over previous
import functools

import jax
import jax.numpy as jnp
from jax import lax
from jax.experimental import pallas as pl
from jax.experimental.pallas import tpu as pltpu
from jax.experimental.pallas import tpu_sc as plsc

VOCAB = 1000000
DIM = 64
BATCH = 16384
CTX = 20
NEG = 20
SCORE = 1 + NEG

NC = 2
NS = 16
NW = NC * NS
B_PER_W = BATCH // NW
R = 8
NCHUNK = B_PER_W // R
CTX_N = R * CTX
OUT_N = R * SCORE
CG = CTX_N // 2
OG = OUT_N // 2
FILL = 1.0e9


def _sc_dots(cat_tbl, ctx3d, outs3d):
    mesh = plsc.VectorSubcoreMesh(core_axis_name="c", subcore_axis_name="s")

    @functools.partial(
        pl.kernel,
        mesh=mesh,
        out_type=jax.ShapeDtypeStruct((BATCH, 32), jnp.float32),
        compiler_params=pltpu.CompilerParams(
            needs_layout_passes=False, use_tc_tiling_on_sc=False),
        scratch_types=[
            pltpu.VMEM((B_PER_W * CTX // CG, CG), jnp.int32),
            pltpu.VMEM((B_PER_W * SCORE // OG, OG), jnp.int32),
            pltpu.VMEM((2, CTX_N, 2 * DIM), jnp.float32),
            pltpu.VMEM((2, OUT_N, 2 * DIM), jnp.float32),
            pltpu.VMEM((R, 32), jnp.float32),
            pltpu.SemaphoreType.DMA,
            pltpu.SemaphoreType.DMA,
        ],
    )
    def k(tbl_hbm, ctx_hbm, outs_hbm, dots_o,
          ctx_idx, outs_idx, ctx_rows, outs_rows, dots_v, sem0, sem1):
        wid = lax.axis_index("s") * NC + lax.axis_index("c")
        lane = lax.iota(jnp.int32, 16)
        sems = (sem0, sem1)
        pltpu.sync_copy(ctx_hbm.at[wid], ctx_idx)
        pltpu.sync_copy(outs_hbm.at[wid], outs_idx)

        def fire(i, buf):
            sem = sems[buf]
            for s in range(2):
                pltpu.async_copy(
                    tbl_hbm.at[ctx_idx.at[2 * i + s]],
                    ctx_rows.at[buf, pl.ds(s * CG, CG)], sem)
                pltpu.async_copy(
                    tbl_hbm.at[outs_idx.at[2 * i + s]],
                    outs_rows.at[buf, pl.ds(s * OG, OG)], sem)

        def drain(buf):
            sem = sems[buf]
            for s in range(2):
                pltpu.make_async_copy(
                    tbl_hbm.at[pl.ds(0, CG)],
                    ctx_rows.at[buf, pl.ds(s * CG, CG)], sem).wait()
                pltpu.make_async_copy(
                    tbl_hbm.at[pl.ds(0, OG)],
                    outs_rows.at[buf, pl.ds(s * OG, OG)], sem).wait()

        def compute(i, buf):
            def row_body(r, _):
                h = []
                for d in range(DIM // 16):
                    acc = ctx_rows[buf, r * CTX, pl.ds(d * 16, 16)]
                    for c in range(1, CTX):
                        acc = acc + ctx_rows[buf, r * CTX + c,
                                             pl.ds(d * 16, 16)]
                    h.append(acc * (1.0 / CTX))
                nh = [-v for v in h]
                v0 = jnp.full((16,), FILL)
                v1 = jnp.full((16,), FILL)
                for j in range(SCORE):
                    hh = h if j == 0 else nh
                    acc = outs_rows[buf, r * SCORE + j, pl.ds(DIM, 16)] * hh[0]
                    for d in range(1, DIM // 16):
                        acc = acc + outs_rows[buf, r * SCORE + j,
                                              pl.ds(DIM + d * 16, 16)] * hh[d]
                    dot = jnp.sum(acc)
                    if j < 16:
                        v0 = jnp.where(lane == j, dot, v0)
                    else:
                        v1 = jnp.where(lane == (j - 16), dot, v1)
                dots_v[r, pl.ds(0, 16)] = v0
                dots_v[r, pl.ds(16, 16)] = v1
                return 0

            lax.fori_loop(0, R, row_body, 0)
            pltpu.sync_copy(dots_v,
                            dots_o.at[pl.ds(wid * B_PER_W + i * R, R)])

        fire(0, 0)

        def pair_body(t, _):
            fire(2 * t + 1, 1)
            drain(0)
            compute(2 * t, 0)

            @pl.when(t < NCHUNK // 2 - 1)
            def _():
                fire(2 * t + 2, 0)

            drain(1)
            compute(2 * t + 1, 1)
            return 0

        lax.fori_loop(0, NCHUNK // 2, pair_body, 0)

    return k(cat_tbl, ctx3d, outs3d)


def _tc_loss(dots2d):
    def body(dots_ref, out_ref):
        s = -jnp.sum(jax.nn.log_sigmoid(dots_ref[...])) / BATCH
        out_ref[...] = jnp.full((1, 1), s, dtype=jnp.float32)

    out = pl.pallas_call(
        body,
        out_shape=jax.ShapeDtypeStruct((1, 1), jnp.float32),
    )(dots2d)
    return out[0, 0]


def kernel(in_table, out_table, contexts, targets, negative_sampling):
    cat_tbl = jnp.concatenate([in_table, out_table], axis=1)
    ctx3d = contexts.astype(jnp.int32).reshape(NW, B_PER_W * CTX // CG, CG)
    outs = jnp.concatenate(
        [targets, negative_sampling], axis=1)
    outs3d = outs.astype(jnp.int32).reshape(NW, B_PER_W * SCORE // OG, OG)
    dots = _sc_dots(cat_tbl, ctx3d, outs3d)
    return _tc_loss(dots.reshape(BATCH * 32 // 128, 128))
